# baseline (speedup 1.0000x reference)
import functools

import jax
import jax.numpy as jnp
from jax import lax
from jax.experimental import pallas as pl
from jax.experimental.pallas import tpu as pltpu
from jax.experimental.pallas import tpu_sc as plsc

_BINS = 10
_L = 16
_NC = 2
_NS = 16
_NW = _NC * _NS

_LOG1P_C = (
    9.97503255e-06, 9.99235484e-01, -4.90230723e-01, 2.85272681e-01,
    -1.31581825e-01, 3.04490045e-02,
)

_SIGP_C = (
    9.99998718, -9.99826672, 9.96083614, -9.65306581, 8.41720693,
    -5.73313265, 2.51346951, -0.50704365,
)

_CW = 128
_GPR = _CW // _L


@functools.lru_cache(maxsize=None)
def _make_hist_kernel(rows_t: int, cols_t: int):
    assert rows_t % 8 == 0
    n_full = cols_t // _CW
    k_uni = n_full // _NW
    n_extra = n_full - k_uni * _NW
    rag_off = n_full * _CW
    rag_cols = cols_t - rag_off
    assert k_uni % 2 == 0 and k_uni >= 2 and n_extra < _NW
    assert rag_cols % _L == 0
    rag_gpr = rag_cols // _L

    mesh = plsc.VectorSubcoreMesh(core_axis_name="c", subcore_axis_name="s")

    @functools.partial(
        pl.kernel,
        out_type=jax.ShapeDtypeStruct((_NW, 2 * _L), jnp.float32),
        mesh=mesh,
        scratch_types=[
            pltpu.VMEM((rows_t, _CW), jnp.float32),
            pltpu.VMEM((rows_t, _CW), jnp.float32),
            pltpu.VMEM((rows_t, _CW), jnp.float32),
            pltpu.VMEM((rows_t, _CW), jnp.float32),
            pltpu.VMEM((rows_t, _CW), jnp.float32),
            pltpu.VMEM((rows_t, _CW), jnp.float32),
            pltpu.VMEM((rows_t, _CW), jnp.float32),
            pltpu.VMEM((rows_t, _CW), jnp.float32),
            pltpu.VMEM((rows_t, _CW), jnp.float32),
        ] + [pltpu.VMEM((_L * _L,), jnp.float32)
             for _ in range(2 * _GPR)
        ] + [
            pltpu.VMEM((2 * _L,), jnp.float32),
            pltpu.SemaphoreType.DMA,
            pltpu.SemaphoreType.DMA,
            pltpu.SemaphoreType.DMA,
        ],
        compiler_params=pltpu.CompilerParams(needs_layout_passes=False),
    )
    def hist_kernel(p_hbm, t_hbm, w_hbm, out_hbm,
                    pb0, pb1, tb0, tb1, wb0, wb1, rp, rt, rw,
                    *rest):
        accws = rest[:_GPR]
        acccs = rest[_GPR:2 * _GPR]
        res, sem0, sem1, sem2 = rest[2 * _GPR:]
        wid = lax.axis_index("s") * _NC + lax.axis_index("c")
        sems = (sem0, sem1)
        pbufs = (pb0, pb1)
        tbufs = (tb0, tb1)
        wbufs = (wb0, wb1)

        zeros = jnp.zeros((_L,), jnp.float32)
        for i in range(_L):
            for g in range(_GPR):
                accws[g][pl.ds(i * _L, _L)] = zeros
                acccs[g][pl.ds(i * _L, _L)] = zeros

        def slab_col(k):
            c = jnp.where(jnp.asarray(k) < k_uni,
                          wid + _NW * jnp.asarray(k),
                          jnp.minimum(k_uni * _NW + wid, n_full - 1))
            return pl.multiple_of(c * _CW, _CW)

        def issue(k, b):
            col0 = slab_col(k)
            pltpu.async_copy(
                p_hbm.at[pl.ds(0, rows_t), pl.ds(col0, _CW)], pbufs[b], sems[b])
            pltpu.async_copy(
                t_hbm.at[pl.ds(0, rows_t), pl.ds(col0, _CW)], tbufs[b], sems[b])
            pltpu.async_copy(
                w_hbm.at[pl.ds(0, rows_t), pl.ds(col0, _CW)], wbufs[b], sems[b])

        def wait_slot(b):
            src = p_hbm.at[pl.ds(0, rows_t), pl.ds(0, _CW)]
            pltpu.make_async_copy(src, pbufs[b], sems[b]).wait()
            pltpu.make_async_copy(src, tbufs[b], sems[b]).wait()
            pltpu.make_async_copy(src, wbufs[b], sems[b]).wait()

        lane = lax.iota(jnp.int32, _L)
        lanebase = lane * _L

        def row_groups(pb, tb, wb, row, groups, scale):
            cols = [lane + g * _L for g in groups]
            ps = [plsc.load_gather(pb, [row, c]) for c in cols]
            ts = [plsc.load_gather(tb, [row, c]) for c in cols]
            ws = [plsc.load_gather(wb, [row, c]) for c in cols]
            if scale is not None:
                ws = [w * scale for w in ws]
            us = [jnp.exp(-jnp.abs(p)) for p in ps]
            hs = [jnp.full((_L,), _SIGP_C[-1], dtype=jnp.float32)
                  for _ in groups]
            for c in _SIGP_C[-2::-1]:
                hs = [h * u + c for h, u in zip(hs, us)]
            gms = [jnp.where((p < 0.0) ^ (t > 0.0), 10.0 - h, h)
                   for p, t, h in zip(ps, ts, hs)]
            idxs = [lanebase + jnp.minimum(gm, 9.0).astype(jnp.int32)
                    for gm in gms]
            c_hi = jnp.full((_L,), _LOG1P_C[-1], dtype=jnp.float32)
            accs = [c_hi for _ in groups]
            for c in _LOG1P_C[-2::-1]:
                accs = [a * u + c for a, u in zip(accs, us)]
            bces = [jnp.maximum(p, 0.0) - p * t + a
                    for p, t, a in zip(ps, ts, accs)]
            for g, idx, bce, w in zip(groups, idxs, bces, ws):
                plsc.addupdate_scatter(accws[g], [idx], bce * w)
                plsc.addupdate_scatter(acccs[g], [idx], w)

        def compute_full(b, scale):
            pb, tb, wb = pbufs[b], tbufs[b], wbufs[b]

            @plsc.parallel_loop(0, rows_t, 1, unroll=1)
            def body(j):
                row = jnp.zeros((_L,), jnp.int32) + j
                row_groups(pb, tb, wb, row, list(range(_GPR)), scale)

        rag0 = pl.multiple_of(rag_off, _CW)
        pltpu.async_copy(
            p_hbm.at[pl.ds(0, rows_t), pl.ds(rag0, _CW)], rp, sem2)
        pltpu.async_copy(
            t_hbm.at[pl.ds(0, rows_t), pl.ds(rag0, _CW)], rt, sem2)
        pltpu.async_copy(
            w_hbm.at[pl.ds(0, rows_t), pl.ds(rag0, _CW)], rw, sem2)

        issue(0, 0)

        def outer(j, carry):
            issue(2 * j + 1, 1)
            wait_slot(0)
            compute_full(0, None)
            issue(2 * j + 2, 0)
            wait_slot(1)
            compute_full(1, None)
            return carry

        lax.fori_loop(0, k_uni // 2, outer, 0)

        wid_v = jnp.zeros((_L,), jnp.int32) + wid
        extra_flag = jnp.where(wid_v < n_extra, 1.0, 0.0)
        wait_slot(0)
        compute_full(0, extra_flag)

        src = p_hbm.at[pl.ds(0, rows_t), pl.ds(0, _CW)]
        pltpu.make_async_copy(src, rp, sem2).wait()
        pltpu.make_async_copy(src, rt, sem2).wait()
        pltpu.make_async_copy(src, rw, sem2).wait()
        rag_flag = jnp.where(wid_v == _NW - 1, 1.0, 0.0)

        @plsc.parallel_loop(0, rows_t, 1, unroll=2)
        def rag_body(j):
            row = jnp.zeros((_L,), jnp.int32) + j
            row_groups(rp, rt, rw, row, list(range(rag_gpr)), rag_flag)

        wv = accws[0][pl.ds(0, _L)]
        cv = acccs[0][pl.ds(0, _L)]
        for g in range(_GPR):
            for l in range(_L):
                if g == 0 and l == 0:
                    continue
                wv = wv + accws[g][pl.ds(l * _L, _L)]
                cv = cv + acccs[g][pl.ds(l * _L, _L)]
        res[pl.ds(0, _L)] = wv
        res[pl.ds(_L, _L)] = cv
        pltpu.sync_copy(res, out_hbm.at[wid])

    return hist_kernel


def kernel(pred, target, weight):
    n_rows, n_cols = pred.shape
    parts = _make_hist_kernel(n_cols, n_rows)(pred.T, target.T, weight.T)
    sums = jnp.sum(parts, axis=0)
    w_b = sums[:_BINS]
    c_b = sums[_L:_L + _BINS]
    nne = jnp.sum((c_b > 0).astype(jnp.float32))
    loss = jnp.sum(jnp.where(c_b > 0, w_b / jnp.maximum(c_b, 1.0), 0.0))
    return loss / jnp.maximum(nne, 1.0)


# speedup vs baseline: 1.4650x; 1.4650x over previous
import functools

import jax
import jax.numpy as jnp
from jax import lax
from jax.experimental import pallas as pl
from jax.experimental.pallas import tpu as pltpu
from jax.experimental.pallas import tpu_sc as plsc

_BINS = 10
_L = 16
_NC = 2
_NS = 16
_NW = _NC * _NS

_LOG1P_C = (
    6.94457445e-05, 9.96261948e-01, -4.66442439e-01, 2.18665484e-01,
    -5.54593137e-02,
)


_CW = 128
_GPR = _CW // _L


@functools.lru_cache(maxsize=None)
def _make_hist_kernel(rows_t: int, cols_t: int):
    assert rows_t % 8 == 0
    n_full = cols_t // _CW
    k_uni = n_full // _NW
    n_extra = n_full - k_uni * _NW
    rag_off = n_full * _CW
    rag_cols = cols_t - rag_off
    assert k_uni % 2 == 0 and k_uni >= 2 and n_extra < _NW
    assert rag_cols % _L == 0
    rag_gpr = rag_cols // _L

    mesh = plsc.VectorSubcoreMesh(core_axis_name="c", subcore_axis_name="s")

    @functools.partial(
        pl.kernel,
        out_type=jax.ShapeDtypeStruct((_NW, 2 * _L), jnp.float32),
        mesh=mesh,
        scratch_types=[
            pltpu.VMEM((rows_t, _CW), jnp.float32),
            pltpu.VMEM((rows_t, _CW), jnp.float32),
            pltpu.VMEM((rows_t, _CW), jnp.float32),
            pltpu.VMEM((rows_t, _CW), jnp.float32),
            pltpu.VMEM((rows_t, _CW), jnp.float32),
            pltpu.VMEM((rows_t, _CW), jnp.float32),
            pltpu.VMEM((rows_t, _CW), jnp.float32),
            pltpu.VMEM((rows_t, _CW), jnp.float32),
            pltpu.VMEM((rows_t, _CW), jnp.float32),
        ] + [pltpu.VMEM((_L * _L,), jnp.float32)
             for _ in range(2 * _GPR)
        ] + [
            pltpu.VMEM((2 * _L,), jnp.float32),
            pltpu.SemaphoreType.DMA,
            pltpu.SemaphoreType.DMA,
            pltpu.SemaphoreType.DMA,
        ],
        compiler_params=pltpu.CompilerParams(needs_layout_passes=False),
    )
    def hist_kernel(p_hbm, t_hbm, w_hbm, out_hbm,
                    pb0, pb1, tb0, tb1, wb0, wb1, rp, rt, rw,
                    *rest):
        accws = rest[:_GPR]
        acccs = rest[_GPR:2 * _GPR]
        res, sem0, sem1, sem2 = rest[2 * _GPR:]
        wid = lax.axis_index("s") * _NC + lax.axis_index("c")
        sems = (sem0, sem1)
        pbufs = (pb0, pb1)
        tbufs = (tb0, tb1)
        wbufs = (wb0, wb1)

        zeros = jnp.zeros((_L,), jnp.float32)
        for i in range(_L):
            for g in range(_GPR):
                accws[g][pl.ds(i * _L, _L)] = zeros
                acccs[g][pl.ds(i * _L, _L)] = zeros

        def slab_col(k):
            c = jnp.where(jnp.asarray(k) < k_uni,
                          wid + _NW * jnp.asarray(k),
                          jnp.minimum(k_uni * _NW + wid, n_full - 1))
            return pl.multiple_of(c * _CW, _CW)

        def issue(k, b):
            col0 = slab_col(k)
            pltpu.async_copy(
                p_hbm.at[pl.ds(0, rows_t), pl.ds(col0, _CW)], pbufs[b], sems[b])
            pltpu.async_copy(
                t_hbm.at[pl.ds(0, rows_t), pl.ds(col0, _CW)], tbufs[b], sems[b])
            pltpu.async_copy(
                w_hbm.at[pl.ds(0, rows_t), pl.ds(col0, _CW)], wbufs[b], sems[b])

        def wait_slot(b):
            src = p_hbm.at[pl.ds(0, rows_t), pl.ds(0, _CW)]
            pltpu.make_async_copy(src, pbufs[b], sems[b]).wait()
            pltpu.make_async_copy(src, tbufs[b], sems[b]).wait()
            pltpu.make_async_copy(src, wbufs[b], sems[b]).wait()

        lane = lax.iota(jnp.int32, _L)
        lanebase = lane * _L

        def row_groups(pb, tb, wb, row, groups, scale):
            cols = [lane + g * _L for g in groups]
            ps = [plsc.load_gather(pb, [row, c]) for c in cols]
            ts = [plsc.load_gather(tb, [row, c]) for c in cols]
            ws = [plsc.load_gather(wb, [row, c]) for c in cols]
            if scale is not None:
                ws = [w * scale for w in ws]
            es = [jnp.exp(-p) for p in ps]
            ss = [1.0 / (1.0 + e) for e in es]
            us = [jnp.minimum(e, 1.0 / e) for e in es]
            gms = [jnp.abs(s - t) for s, t in zip(ss, ts)]
            idxs = [lanebase
                    + jnp.minimum(gm * 10.0, 9.0).astype(jnp.int32)
                    for gm in gms]
            c_hi = jnp.full((_L,), _LOG1P_C[-1], dtype=jnp.float32)
            accs = [c_hi for _ in groups]
            for c in _LOG1P_C[-2::-1]:
                accs = [a * u + c for a, u in zip(accs, us)]
            bces = [jnp.maximum(p, 0.0) - p * t + a
                    for p, t, a in zip(ps, ts, accs)]
            for g, idx, bce, w in zip(groups, idxs, bces, ws):
                plsc.addupdate_scatter(accws[g], [idx], bce * w)
                plsc.addupdate_scatter(acccs[g], [idx], w)

        def compute_full(b, scale):
            pb, tb, wb = pbufs[b], tbufs[b], wbufs[b]

            @plsc.parallel_loop(0, rows_t, 1, unroll=1)
            def body(j):
                row = jnp.zeros((_L,), jnp.int32) + j
                row_groups(pb, tb, wb, row, list(range(_GPR)), scale)

        rag0 = pl.multiple_of(rag_off, _CW)
        pltpu.async_copy(
            p_hbm.at[pl.ds(0, rows_t), pl.ds(rag0, _CW)], rp, sem2)
        pltpu.async_copy(
            t_hbm.at[pl.ds(0, rows_t), pl.ds(rag0, _CW)], rt, sem2)
        pltpu.async_copy(
            w_hbm.at[pl.ds(0, rows_t), pl.ds(rag0, _CW)], rw, sem2)

        issue(0, 0)

        def outer(j, carry):
            issue(2 * j + 1, 1)
            wait_slot(0)
            compute_full(0, None)
            issue(2 * j + 2, 0)
            wait_slot(1)
            compute_full(1, None)
            return carry

        lax.fori_loop(0, k_uni // 2, outer, 0)

        wid_v = jnp.zeros((_L,), jnp.int32) + wid
        extra_flag = jnp.where(wid_v < n_extra, 1.0, 0.0)
        wait_slot(0)
        compute_full(0, extra_flag)

        src = p_hbm.at[pl.ds(0, rows_t), pl.ds(0, _CW)]
        pltpu.make_async_copy(src, rp, sem2).wait()
        pltpu.make_async_copy(src, rt, sem2).wait()
        pltpu.make_async_copy(src, rw, sem2).wait()
        rag_flag = jnp.where(wid_v == _NW - 1, 1.0, 0.0)

        @plsc.parallel_loop(0, rows_t, 1, unroll=2)
        def rag_body(j):
            row = jnp.zeros((_L,), jnp.int32) + j
            row_groups(rp, rt, rw, row, list(range(rag_gpr)), rag_flag)

        wv = accws[0][pl.ds(0, _L)]
        cv = acccs[0][pl.ds(0, _L)]
        for g in range(_GPR):
            for l in range(_L):
                if g == 0 and l == 0:
                    continue
                wv = wv + accws[g][pl.ds(l * _L, _L)]
                cv = cv + acccs[g][pl.ds(l * _L, _L)]
        res[pl.ds(0, _L)] = wv
        res[pl.ds(_L, _L)] = cv
        pltpu.sync_copy(res, out_hbm.at[wid])

    return hist_kernel


def kernel(pred, target, weight):
    n_rows, n_cols = pred.shape
    parts = _make_hist_kernel(n_cols, n_rows)(pred.T, target.T, weight.T)
    sums = jnp.sum(parts, axis=0)
    w_b = sums[:_BINS]
    c_b = sums[_L:_L + _BINS]
    nne = jnp.sum((c_b > 0).astype(jnp.float32))
    loss = jnp.sum(jnp.where(c_b > 0, w_b / jnp.maximum(c_b, 1.0), 0.0))
    return loss / jnp.maximum(nne, 1.0)
